# Initial kernel scaffold; baseline (speedup 1.0000x reference)
import jax
import jax.numpy as jnp
from jax.experimental import pallas as pl


def kernel(x, edge_index, W_self, W_neigh, b):
    raise NotImplementedError("write your pallas kernel here")


# speedup vs baseline: 4.8653x; 4.8653x over previous
import jax
import jax.numpy as jnp
from jax import lax
from jax.experimental import pallas as pl
from jax.experimental.pallas import tpu as pltpu
from jax.experimental.pallas import tpu_sc as plsc

N_NODES = 10000
N_EDGES = 320000
D = 128

NC = 2
NS = 16
NW = NC * NS
PW = N_EDGES // NW
CH = 80
NCH = PW // CH

_sc_mesh = plsc.VectorSubcoreMesh(core_axis_name="c", subcore_axis_name="s")


def _sc_body(x_hbm, src_hbm, dst_hbm, acc_hbm, deg_hbm,
             src_v, dst_v, rows_v, acc_sh, sem):
    cid = lax.axis_index("c")
    sid = lax.axis_index("s")
    wid = sid * NC + cid

    zero16 = jnp.zeros((16,), jnp.float32)
    ones16 = jnp.ones((16,), jnp.float32)

    def _fill_rows(val16):
        def _f(r, _):
            for j in range(D // 16):
                rows_v[r, pl.ds(j * 16, 16)] = val16
            return 0
        lax.fori_loop(0, CH, _f, 0)

    n_row_chunks = N_NODES // CH
    my_chunks = jnp.where(sid < n_row_chunks % NS,
                          n_row_chunks // NS + 1, n_row_chunks // NS)

    def _zero_acc(j, _):
        off = (sid + j * NS) * CH
        pltpu.sync_copy(rows_v, acc_sh.at[pl.ds(off, CH)])
        return 0

    def _writeback_to(out_hbm):
        def _wb(j, _):
            off = (sid + j * NS) * CH
            pltpu.sync_copy(acc_sh.at[pl.ds(off, CH)], rows_v)
            pltpu.sync_copy(rows_v, out_hbm.at[cid].at[pl.ds(off, CH)])
            return 0
        lax.fori_loop(0, my_chunks, _wb, 0)

    e0 = wid * PW

    _fill_rows(zero16)
    lax.fori_loop(0, my_chunks, _zero_acc, 0)
    plsc.subcore_barrier()

    def _edge_chunk(c, _):
        pltpu.sync_copy(src_hbm.at[pl.ds(e0 + c * CH, CH)], src_v)
        pltpu.sync_copy(dst_hbm.at[pl.ds(e0 + c * CH, CH)], dst_v)
        pltpu.async_copy(x_hbm.at[src_v], rows_v, sem).wait()
        pltpu.sync_copy(rows_v, acc_sh.at[dst_v], add=True)
        return 0
    lax.fori_loop(0, NCH, _edge_chunk, 0)

    plsc.subcore_barrier()
    _writeback_to(acc_hbm)
    plsc.subcore_barrier()

    _fill_rows(zero16)
    lax.fori_loop(0, my_chunks, _zero_acc, 0)
    _fill_rows(ones16)
    plsc.subcore_barrier()

    def _deg_chunk(c, _):
        pltpu.sync_copy(dst_hbm.at[pl.ds(e0 + c * CH, CH)], dst_v)
        pltpu.sync_copy(rows_v, acc_sh.at[dst_v], add=True)
        return 0
    lax.fori_loop(0, NCH, _deg_chunk, 0)

    plsc.subcore_barrier()
    _writeback_to(deg_hbm)


_sc_aggregate = pl.kernel(
    _sc_body,
    out_type=[
        jax.ShapeDtypeStruct((NC, N_NODES, D), jnp.float32),
        jax.ShapeDtypeStruct((NC, N_NODES, D), jnp.float32),
    ],
    mesh=_sc_mesh,
    scratch_types=[
        pltpu.VMEM((CH,), jnp.int32),
        pltpu.VMEM((CH,), jnp.int32),
        pltpu.VMEM((CH, D), jnp.float32),
        pltpu.VMEM_SHARED((N_NODES, D), jnp.float32),
        pltpu.SemaphoreType.DMA,
    ],
)


_TC_BLK = 1000


def _tc_body(x_ref, acc_ref, deg_ref, ws_ref, wn_ref, b_ref, o_ref):
    s = acc_ref[0, :, :] + acc_ref[1, :, :]
    deg = jnp.sum(deg_ref[0, :, :] + deg_ref[1, :, :], axis=1) * (1.0 / D)
    mean = s / jnp.maximum(deg, 1.0)[:, None]
    o_ref[...] = jnp.maximum(
        jnp.dot(x_ref[...], ws_ref[...], preferred_element_type=jnp.float32)
        + jnp.dot(mean, wn_ref[...], preferred_element_type=jnp.float32)
        + b_ref[...],
        0.0,
    )


def _tc_finish(x, acc, deg, W_self, W_neigh, b2d):
    grid = (N_NODES // _TC_BLK,)
    return pl.pallas_call(
        _tc_body,
        grid=grid,
        in_specs=[
            pl.BlockSpec((_TC_BLK, D), lambda i: (i, 0)),
            pl.BlockSpec((NC, _TC_BLK, D), lambda i: (0, i, 0)),
            pl.BlockSpec((NC, _TC_BLK, D), lambda i: (0, i, 0)),
            pl.BlockSpec((D, D), lambda i: (0, 0)),
            pl.BlockSpec((D, D), lambda i: (0, 0)),
            pl.BlockSpec((1, D), lambda i: (0, 0)),
        ],
        out_specs=pl.BlockSpec((_TC_BLK, D), lambda i: (i, 0)),
        out_shape=jax.ShapeDtypeStruct((N_NODES, D), jnp.float32),
    )(x, acc, deg, W_self, W_neigh, b2d)


@jax.jit
def kernel(x, edge_index, W_self, W_neigh, b):
    src = edge_index[0].astype(jnp.int32)
    dst = edge_index[1].astype(jnp.int32)
    acc, deg = _sc_aggregate(x, src, dst)
    return _tc_finish(x, acc, deg, W_self, W_neigh, b.reshape(1, D))


# speedup vs baseline: 6.7059x; 1.3783x over previous
import jax
import jax.numpy as jnp
from jax import lax
from jax.experimental import pallas as pl
from jax.experimental.pallas import tpu as pltpu
from jax.experimental.pallas import tpu_sc as plsc

N_NODES = 10000
N_EDGES = 320000
D = 128

NC = 2
NS = 16
NW = NC * NS
PW = N_EDGES // NW
CH = 80
NCH = PW // CH

_sc_mesh = plsc.VectorSubcoreMesh(core_axis_name="c", subcore_axis_name="s")


def _sc_body(x_hbm, src_hbm, dst_hbm, acc_hbm, deg_hbm,
             src_v, dst_v, rows_v, src_b, dst_b, rows_b, acc_sh, sem, sem_b):
    cid = lax.axis_index("c")
    sid = lax.axis_index("s")
    wid = sid * NC + cid

    zero16 = jnp.zeros((16,), jnp.float32)
    ones16 = jnp.ones((16,), jnp.float32)

    def _fill_rows(val16):
        def _f(r, _):
            for j in range(D // 16):
                rows_v[r, pl.ds(j * 16, 16)] = val16
            return 0
        lax.fori_loop(0, CH, _f, 0)

    n_row_chunks = N_NODES // CH
    my_chunks = jnp.where(sid < n_row_chunks % NS,
                          n_row_chunks // NS + 1, n_row_chunks // NS)

    def _zero_acc(j, _):
        off = (sid + j * NS) * CH
        pltpu.sync_copy(rows_v, acc_sh.at[pl.ds(off, CH)])
        return 0

    def _writeback_to(out_hbm):
        def _wb(j, _):
            off = (sid + j * NS) * CH
            pltpu.sync_copy(acc_sh.at[pl.ds(off, CH)], rows_v)
            pltpu.sync_copy(rows_v, out_hbm.at[cid].at[pl.ds(off, CH)])
            return 0
        lax.fori_loop(0, my_chunks, _wb, 0)

    e0 = wid * PW

    _fill_rows(zero16)
    lax.fori_loop(0, my_chunks, _zero_acc, 0)
    plsc.subcore_barrier()

    def _fire(c, s_v, r_v, sm):
        pltpu.sync_copy(src_hbm.at[pl.ds(e0 + c * CH, CH)], s_v)
        pltpu.async_copy(x_hbm.at[s_v], r_v, sm)

    def _drain_scatter(c, s_v, d_v, r_v, sm):
        pltpu.sync_copy(dst_hbm.at[pl.ds(e0 + c * CH, CH)], d_v)
        pltpu.make_async_copy(x_hbm.at[s_v], r_v, sm).wait()
        pltpu.sync_copy(r_v, acc_sh.at[d_v], add=True)

    _fire(0, src_v, rows_v, sem)

    def _edge_pair(i, _):
        c0 = 2 * i
        _fire(c0 + 1, src_b, rows_b, sem_b)
        _drain_scatter(c0, src_v, dst_v, rows_v, sem)

        @pl.when(c0 + 2 < NCH)
        def _():
            _fire(c0 + 2, src_v, rows_v, sem)
        _drain_scatter(c0 + 1, src_b, dst_b, rows_b, sem_b)
        return 0
    lax.fori_loop(0, NCH // 2, _edge_pair, 0)
    if NCH % 2:
        _drain_scatter(NCH - 1, src_v, dst_v, rows_v, sem)

    plsc.subcore_barrier()
    _writeback_to(acc_hbm)
    plsc.subcore_barrier()

    _fill_rows(zero16)
    lax.fori_loop(0, my_chunks, _zero_acc, 0)
    _fill_rows(ones16)
    plsc.subcore_barrier()

    def _deg_chunk(c, _):
        pltpu.sync_copy(dst_hbm.at[pl.ds(e0 + c * CH, CH)], dst_v)
        pltpu.sync_copy(rows_v, acc_sh.at[dst_v], add=True)
        return 0
    lax.fori_loop(0, NCH, _deg_chunk, 0)

    plsc.subcore_barrier()
    _writeback_to(deg_hbm)


_sc_aggregate = pl.kernel(
    _sc_body,
    out_type=[
        jax.ShapeDtypeStruct((NC, N_NODES, D), jnp.float32),
        jax.ShapeDtypeStruct((NC, N_NODES, D), jnp.float32),
    ],
    mesh=_sc_mesh,
    scratch_types=[
        pltpu.VMEM((CH,), jnp.int32),
        pltpu.VMEM((CH,), jnp.int32),
        pltpu.VMEM((CH, D), jnp.float32),
        pltpu.VMEM((CH,), jnp.int32),
        pltpu.VMEM((CH,), jnp.int32),
        pltpu.VMEM((CH, D), jnp.float32),
        pltpu.VMEM_SHARED((N_NODES, D), jnp.float32),
        pltpu.SemaphoreType.DMA,
        pltpu.SemaphoreType.DMA,
    ],
)


_TC_BLK = 1000


def _tc_body(x_ref, acc_ref, deg_ref, ws_ref, wn_ref, b_ref, o_ref):
    s = acc_ref[0, :, :] + acc_ref[1, :, :]
    deg = jnp.sum(deg_ref[0, :, :] + deg_ref[1, :, :], axis=1) * (1.0 / D)
    mean = s / jnp.maximum(deg, 1.0)[:, None]
    o_ref[...] = jnp.maximum(
        jnp.dot(x_ref[...], ws_ref[...], preferred_element_type=jnp.float32)
        + jnp.dot(mean, wn_ref[...], preferred_element_type=jnp.float32)
        + b_ref[...],
        0.0,
    )


def _tc_finish(x, acc, deg, W_self, W_neigh, b2d):
    grid = (N_NODES // _TC_BLK,)
    return pl.pallas_call(
        _tc_body,
        grid=grid,
        in_specs=[
            pl.BlockSpec((_TC_BLK, D), lambda i: (i, 0)),
            pl.BlockSpec((NC, _TC_BLK, D), lambda i: (0, i, 0)),
            pl.BlockSpec((NC, _TC_BLK, D), lambda i: (0, i, 0)),
            pl.BlockSpec((D, D), lambda i: (0, 0)),
            pl.BlockSpec((D, D), lambda i: (0, 0)),
            pl.BlockSpec((1, D), lambda i: (0, 0)),
        ],
        out_specs=pl.BlockSpec((_TC_BLK, D), lambda i: (i, 0)),
        out_shape=jax.ShapeDtypeStruct((N_NODES, D), jnp.float32),
    )(x, acc, deg, W_self, W_neigh, b2d)


@jax.jit
def kernel(x, edge_index, W_self, W_neigh, b):
    src = edge_index[0].astype(jnp.int32)
    dst = edge_index[1].astype(jnp.int32)
    acc, deg = _sc_aggregate(x, src, dst)
    return _tc_finish(x, acc, deg, W_self, W_neigh, b.reshape(1, D))
